# baseline (speedup 1.0000x reference)
import functools

import jax
import jax.numpy as jnp
from jax import lax
from jax.experimental import pallas as pl
from jax.experimental.pallas import tpu as pltpu
from jax.experimental.pallas import tpu_sc as plsc

D = 128
E = 320000

NC = 2
NS = 16
L = 16
NW = NC * NS
EPW = E // NW
K = 80
NCHUNK = EPW // K
NGROUP = K // L


DUNROLL = 16
NACC = 4


def _dot_chunk(u_buf, i_buf, out_v, out_base):
    iota = lax.iota(jnp.int32, L)

    def group_body(g, _):
        e_ids = g * L + iota

        def d_step(t, carry):
            accs = list(carry[:NACC])
            dv = carry[NACC]
            for k in range(DUNROLL):
                dvk = dv + k
                u = plsc.load_gather(u_buf, [e_ids, dvk])
                v = plsc.load_gather(i_buf, [e_ids, dvk])
                accs[k % NACC] = accs[k % NACC] + u * v
            return (*accs, dv + DUNROLL)

        zero = jnp.zeros((L,), jnp.float32)
        dv0 = jnp.zeros((L,), jnp.int32)
        carry = lax.fori_loop(0, D // DUNROLL, d_step, (zero,) * NACC + (dv0,))
        accs = carry[:NACC]
        out_v[pl.ds(out_base + g * L, L)] = (
            (accs[0] + accs[1]) + (accs[2] + accs[3]))
        return 0

    lax.fori_loop(0, NGROUP, group_body, 0)


def _sc_body(user_hbm, item_hbm, src_hbm, dst_hbm, out_hbm,
             idx_u, idx_i, out_v,
             u_a, i_a, u_b, i_b,
             sem_ua, sem_ia, sem_ub, sem_ib):
    wid = lax.axis_index("s") * NC + lax.axis_index("c")
    base = wid * EPW

    pltpu.sync_copy(src_hbm.at[pl.ds(base, EPW)], idx_u)
    pltpu.sync_copy(dst_hbm.at[pl.ds(base, EPW)], idx_i)

    def fire(c, u_dst, i_dst, s_u, s_i):
        off = c * K
        pltpu.async_copy(user_hbm.at[idx_u.at[pl.ds(off, K)]], u_dst, s_u)
        pltpu.async_copy(item_hbm.at[idx_i.at[pl.ds(off, K)]], i_dst, s_i)

    def drain(u_dst, i_dst, s_u, s_i):
        pltpu.make_async_copy(user_hbm.at[idx_u.at[pl.ds(0, K)]], u_dst, s_u).wait()
        pltpu.make_async_copy(item_hbm.at[idx_i.at[pl.ds(0, K)]], i_dst, s_i).wait()

    fire(0, u_a, i_a, sem_ua, sem_ia)

    def pair_body(cc, _):
        a = cc * 2
        b = a + 1
        fire(b, u_b, i_b, sem_ub, sem_ib)
        drain(u_a, i_a, sem_ua, sem_ia)
        _dot_chunk(u_a, i_a, out_v, a * K)
        fire(a + 2, u_a, i_a, sem_ua, sem_ia)
        drain(u_b, i_b, sem_ub, sem_ib)
        _dot_chunk(u_b, i_b, out_v, b * K)
        return 0

    lax.fori_loop(0, (NCHUNK - 1) // 2, pair_body, 0)
    drain(u_a, i_a, sem_ua, sem_ia)
    _dot_chunk(u_a, i_a, out_v, (NCHUNK - 1) * K)

    pltpu.sync_copy(out_v, out_hbm.at[pl.ds(base, EPW)])


@jax.jit
def _run(user_embedding, item_embedding, src, dst):
    mesh = plsc.VectorSubcoreMesh(core_axis_name="c", subcore_axis_name="s")
    kern = pl.kernel(
        _sc_body,
        out_type=jax.ShapeDtypeStruct((E,), jnp.float32),
        mesh=mesh,
        compiler_params=pltpu.CompilerParams(needs_layout_passes=False),
        scratch_types=[
            pltpu.VMEM((EPW,), jnp.int32),
            pltpu.VMEM((EPW,), jnp.int32),
            pltpu.VMEM((EPW,), jnp.float32),
            pltpu.VMEM((K, D), jnp.float32),
            pltpu.VMEM((K, D), jnp.float32),
            pltpu.VMEM((K, D), jnp.float32),
            pltpu.VMEM((K, D), jnp.float32),
            pltpu.SemaphoreType.DMA,
            pltpu.SemaphoreType.DMA,
            pltpu.SemaphoreType.DMA,
            pltpu.SemaphoreType.DMA,
        ],
    )
    return kern(user_embedding, item_embedding, src, dst)


def kernel(user_embedding, item_embedding, edge_index):
    ei = edge_index.astype(jnp.int32)
    return _run(user_embedding, item_embedding, ei[0], ei[1])


# speedup vs baseline: 6.9231x; 6.9231x over previous
import functools

import jax
import jax.numpy as jnp
from jax import lax
from jax.experimental import pallas as pl
from jax.experimental.pallas import tpu as pltpu
from jax.experimental.pallas import tpu_sc as plsc

D = 128
E = 320000

NC = 2
NS = 16
L = 16
NW = NC * NS
EPW = E // NW
K = 80
NCHUNK = EPW // K
NGROUP = K // L


DUNROLL = 16
NACC = 4


def _dot_chunk(u_buf, i_buf, out_v, out_base):
    iota = lax.iota(jnp.int32, L)
    rots = [jnp.bitwise_and(iota + k, L - 1) for k in range(DUNROLL)]

    def group_body(g, _):
        e_ids = g * L + iota

        def d_step(t, carry):
            accs = list(carry[:NACC])
            dv = carry[NACC]
            for k in range(DUNROLL):
                dvk = dv + rots[k]
                u = plsc.load_gather(u_buf, [e_ids, dvk])
                v = plsc.load_gather(i_buf, [e_ids, dvk])
                accs[k % NACC] = accs[k % NACC] + u * v
            return (*accs, dv + DUNROLL)

        zero = jnp.zeros((L,), jnp.float32)
        dv0 = jnp.zeros((L,), jnp.int32)
        carry = lax.fori_loop(0, D // DUNROLL, d_step, (zero,) * NACC + (dv0,))
        accs = carry[:NACC]
        out_v[pl.ds(out_base + g * L, L)] = (
            (accs[0] + accs[1]) + (accs[2] + accs[3]))
        return 0

    lax.fori_loop(0, NGROUP, group_body, 0)


def _sc_body(user_hbm, item_hbm, src_hbm, dst_hbm, out_hbm,
             idx_u, idx_i, out_v,
             u_a, i_a, u_b, i_b,
             sem_ua, sem_ia, sem_ub, sem_ib):
    wid = lax.axis_index("s") * NC + lax.axis_index("c")
    base = wid * EPW

    pltpu.sync_copy(src_hbm.at[pl.ds(base, EPW)], idx_u)
    pltpu.sync_copy(dst_hbm.at[pl.ds(base, EPW)], idx_i)

    def fire(c, u_dst, i_dst, s_u, s_i):
        off = c * K
        pltpu.async_copy(user_hbm.at[idx_u.at[pl.ds(off, K)]], u_dst, s_u)
        pltpu.async_copy(item_hbm.at[idx_i.at[pl.ds(off, K)]], i_dst, s_i)

    def drain(u_dst, i_dst, s_u, s_i):
        pltpu.make_async_copy(user_hbm.at[idx_u.at[pl.ds(0, K)]], u_dst, s_u).wait()
        pltpu.make_async_copy(item_hbm.at[idx_i.at[pl.ds(0, K)]], i_dst, s_i).wait()

    fire(0, u_a, i_a, sem_ua, sem_ia)

    def pair_body(cc, _):
        a = cc * 2
        b = a + 1
        fire(b, u_b, i_b, sem_ub, sem_ib)
        drain(u_a, i_a, sem_ua, sem_ia)
        _dot_chunk(u_a, i_a, out_v, a * K)
        fire(a + 2, u_a, i_a, sem_ua, sem_ia)
        drain(u_b, i_b, sem_ub, sem_ib)
        _dot_chunk(u_b, i_b, out_v, b * K)
        return 0

    lax.fori_loop(0, (NCHUNK - 1) // 2, pair_body, 0)
    drain(u_a, i_a, sem_ua, sem_ia)
    _dot_chunk(u_a, i_a, out_v, (NCHUNK - 1) * K)

    pltpu.sync_copy(out_v, out_hbm.at[pl.ds(base, EPW)])


@jax.jit
def _run(user_embedding, item_embedding, src, dst):
    mesh = plsc.VectorSubcoreMesh(core_axis_name="c", subcore_axis_name="s")
    kern = pl.kernel(
        _sc_body,
        out_type=jax.ShapeDtypeStruct((E,), jnp.float32),
        mesh=mesh,
        compiler_params=pltpu.CompilerParams(needs_layout_passes=False),
        scratch_types=[
            pltpu.VMEM((EPW,), jnp.int32),
            pltpu.VMEM((EPW,), jnp.int32),
            pltpu.VMEM((EPW,), jnp.float32),
            pltpu.VMEM((K, D), jnp.float32),
            pltpu.VMEM((K, D), jnp.float32),
            pltpu.VMEM((K, D), jnp.float32),
            pltpu.VMEM((K, D), jnp.float32),
            pltpu.SemaphoreType.DMA,
            pltpu.SemaphoreType.DMA,
            pltpu.SemaphoreType.DMA,
            pltpu.SemaphoreType.DMA,
        ],
    )
    return kern(user_embedding, item_embedding, src, dst)


def kernel(user_embedding, item_embedding, edge_index):
    ei = edge_index.astype(jnp.int32)
    return _run(user_embedding, item_embedding, ei[0], ei[1])
